# Initial kernel scaffold; baseline (speedup 1.0000x reference)
import jax
import jax.numpy as jnp
from jax.experimental import pallas as pl


def kernel(x, edge_index, batch, W1, b1, W2, b2, Wlin, blin):
    raise NotImplementedError("write your pallas kernel here")


# speedup vs baseline: 29.3724x; 29.3724x over previous
import jax
import jax.numpy as jnp
from jax import lax
from jax.experimental import pallas as pl
from jax.experimental.pallas import tpu as pltpu
from jax.experimental.pallas import tpu_sc as plsc

N = 10000
E = 320000
D = 128
NCLS = 16
NG = 64

NC = 2
NS = 16
NW = NC * NS
EPT = E // NW
CH = 80
NCH = EPT // CH
N_PAD = 10240
RPT = N_PAD // NS
DDEG = 16

_mesh = plsc.VectorSubcoreMesh(core_axis_name="c", subcore_axis_name="s")


def _sc_degree(dst_t):
    @pl.kernel(
        out_type=jax.ShapeDtypeStruct((NC, N_PAD, DDEG), jnp.float32),
        mesh=_mesh,
        compiler_params=pltpu.CompilerParams(use_tc_tiling_on_sc=False),
        scratch_types=[
            pltpu.VMEM((NCH, CH), jnp.int32),
            pltpu.VMEM((CH, DDEG), jnp.float32),
            pltpu.VMEM_SHARED((N_PAD, DDEG), jnp.float32),
            pltpu.SemaphoreType.DMA,
        ],
    )
    def k(dst_hbm, out_hbm, dst_v, ones_v, acc_sh, sem):
        cid = lax.axis_index("c")
        sid = lax.axis_index("s")
        wid = cid * NS + sid
        pltpu.sync_copy(dst_hbm.at[wid], dst_v)

        zero16 = jnp.zeros((16,), jnp.float32)

        @pl.loop(0, CH)
        def _(r):
            ones_v[r, :] = zero16

        @pl.loop(0, RPT // CH)
        def _(i):
            pltpu.sync_copy(ones_v, acc_sh.at[pl.ds(sid * RPT + i * CH, CH)])

        one16 = jnp.full((16,), 1.0, jnp.float32)

        @pl.loop(0, CH)
        def _(r):
            ones_v[r, :] = one16

        plsc.subcore_barrier()

        @pl.loop(0, NCH)
        def _(j):
            pltpu.sync_copy(ones_v, acc_sh.at[dst_v.at[j]], add=True)

        plsc.subcore_barrier()
        pltpu.sync_copy(acc_sh.at[pl.ds(sid * RPT, RPT)],
                        out_hbm.at[cid, pl.ds(sid * RPT, RPT)])

    return k(dst_t)


def _sc_aggregate(hs, src_t, dst_t):
    @pl.kernel(
        out_type=jax.ShapeDtypeStruct((NC, N_PAD, D), jnp.float32),
        mesh=_mesh,
        compiler_params=pltpu.CompilerParams(use_tc_tiling_on_sc=False),
        scratch_types=[
            pltpu.VMEM((NCH, CH), jnp.int32),
            pltpu.VMEM((NCH, CH), jnp.int32),
            pltpu.VMEM((CH, D), jnp.float32),
            pltpu.VMEM((CH, D), jnp.float32),
            pltpu.VMEM_SHARED((N_PAD, D), jnp.float32),
            pltpu.SemaphoreType.DMA,
            pltpu.SemaphoreType.DMA,
        ],
    )
    def k(hs_hbm, src_hbm, dst_hbm, out_hbm,
          src_v, dst_v, rows_a, rows_b, acc_sh, sem_a, sem_b):
        cid = lax.axis_index("c")
        sid = lax.axis_index("s")
        wid = cid * NS + sid
        pltpu.sync_copy(src_hbm.at[wid], src_v)
        pltpu.sync_copy(dst_hbm.at[wid], dst_v)

        zero16 = jnp.zeros((16,), jnp.float32)

        @pl.loop(0, CH)
        def _(r):
            @pl.loop(0, D // 16)
            def _(c):
                rows_a[r, pl.ds(c * 16, 16)] = zero16

        @pl.loop(0, RPT // CH)
        def _(i):
            pltpu.sync_copy(rows_a, acc_sh.at[pl.ds(sid * RPT + i * CH, CH)])

        plsc.subcore_barrier()

        def gather(j, buf, sem):
            pltpu.async_copy(hs_hbm.at[src_v.at[j]], buf, sem)

        def wait(j, buf, sem):
            pltpu.make_async_copy(hs_hbm.at[src_v.at[j]], buf, sem).wait()

        def scat(j, buf):
            pltpu.sync_copy(buf, acc_sh.at[dst_v.at[j]], add=True)

        gather(0, rows_a, sem_a)

        @pl.loop(0, NCH - 2, step=2)
        def _(j):
            gather(j + 1, rows_b, sem_b)
            wait(j, rows_a, sem_a)
            scat(j, rows_a)
            gather(j + 2, rows_a, sem_a)
            wait(j + 1, rows_b, sem_b)
            scat(j + 1, rows_b)

        wait(NCH - 1, rows_a, sem_a)
        scat(NCH - 1, rows_a)

        plsc.subcore_barrier()
        pltpu.sync_copy(acc_sh.at[pl.ds(sid * RPT, RPT)],
                        out_hbm.at[cid, pl.ds(sid * RPT, RPT)])

    return k(hs, src_t, dst_t)


def _tc_matmul(x, W):
    def body(x_ref, w_ref, o_ref):
        o_ref[...] = jnp.dot(x_ref[...], w_ref[...],
                             preferred_element_type=jnp.float32)
    return pl.pallas_call(
        body, out_shape=jax.ShapeDtypeStruct((x.shape[0], W.shape[1]),
                                             jnp.float32))(x, W)


def _tc_scale(h, d0, d1):
    def body(h_ref, d0_ref, d1_ref, hs_ref, dinv_ref):
        dinv = lax.rsqrt(1.0 + d0_ref[...] + d1_ref[...])
        dinv_ref[...] = dinv
        hs_ref[...] = h_ref[...] * dinv
    return pl.pallas_call(
        body,
        out_shape=(jax.ShapeDtypeStruct((N, D), jnp.float32),
                   jax.ShapeDtypeStruct((N, 1), jnp.float32)))(h, d0, d1)


def _tc_layer_out(p0, p1, hs, dinv, b, W):
    def body(p0_ref, p1_ref, hs_ref, dinv_ref, b_ref, w_ref, o_ref):
        agg = dinv_ref[...] * (p0_ref[...] + p1_ref[...] + hs_ref[...])
        out = jnp.maximum(agg + b_ref[...], 0.0)
        o_ref[...] = dinv_ref[...] * jnp.dot(out, w_ref[...],
                                             preferred_element_type=jnp.float32)
    return pl.pallas_call(
        body, out_shape=jax.ShapeDtypeStruct((N, D), jnp.float32))(
            p0, p1, hs, dinv, b, W)


def _tc_head(p0, p1, hs, dinv, b, batch2d, Wlin, blin):
    def body(p0_ref, p1_ref, hs_ref, dinv_ref, b_ref, batch_ref,
             wl_ref, bl_ref, o_ref):
        agg = dinv_ref[...] * (p0_ref[...] + p1_ref[...] + hs_ref[...])
        out = jnp.maximum(agg + b_ref[...], 0.0)
        gids = lax.broadcasted_iota(jnp.int32, (NG, N), 0)
        P = (gids == batch_ref[...]).astype(jnp.float32)
        sums = jnp.dot(P, out, preferred_element_type=jnp.float32)
        counts = jnp.sum(P, axis=1, keepdims=True)
        pooled = sums / jnp.maximum(counts, 1.0)
        o_ref[...] = jnp.dot(pooled, wl_ref[...],
                             preferred_element_type=jnp.float32) + bl_ref[...]
    return pl.pallas_call(
        body, out_shape=jax.ShapeDtypeStruct((NG, NCLS), jnp.float32))(
            p0, p1, hs, dinv, b, batch2d, Wlin, blin)


@jax.jit
def kernel(x, edge_index, batch, W1, b1, W2, b2, Wlin, blin):
    src_t = edge_index[0].reshape(NW, NCH, CH)
    dst_t = edge_index[1].reshape(NW, NCH, CH)
    batch2d = batch.reshape(1, N)
    b1r = b1.reshape(1, D)
    b2r = b2.reshape(1, D)
    blr = blin.reshape(1, NCLS)

    degp = _sc_degree(dst_t)
    h1 = _tc_matmul(x, W1)
    d0 = degp[0, :N, 0:1]
    d1 = degp[1, :N, 0:1]
    hs1, dinv = _tc_scale(h1, d0, d1)

    p = _sc_aggregate(hs1, src_t, dst_t)
    hs2 = _tc_layer_out(p[0, :N], p[1, :N], hs1, dinv, b1r, W2)

    q = _sc_aggregate(hs2, src_t, dst_t)
    return _tc_head(q[0, :N], q[1, :N], hs2, dinv, b2r, batch2d, Wlin, blin)


# speedup vs baseline: 31.2528x; 1.0640x over previous
import jax
import jax.numpy as jnp
from jax import lax
from jax.experimental import pallas as pl
from jax.experimental.pallas import tpu as pltpu
from jax.experimental.pallas import tpu_sc as plsc

N = 10000
E = 320000
D = 128
NCLS = 16
NG = 64

NC = 2
NS = 16
NW = NC * NS
EPT = E // NW
CH = 80
NCH = EPT // CH
N_PAD = 10240
RPT = N_PAD // NS
DDEG = 16

_mesh = plsc.VectorSubcoreMesh(core_axis_name="c", subcore_axis_name="s")
_sc_params = pltpu.CompilerParams(use_tc_tiling_on_sc=False)


def _sc_degree(dst_t):
    @pl.kernel(
        out_type=jax.ShapeDtypeStruct((NC, N_PAD, DDEG), jnp.float32),
        mesh=_mesh,
        compiler_params=_sc_params,
        scratch_types=[
            pltpu.VMEM((NCH, CH), jnp.int32),
            pltpu.VMEM((CH, DDEG), jnp.float32),
            pltpu.VMEM_SHARED((N_PAD, DDEG), jnp.float32),
            pltpu.SemaphoreType.DMA,
        ],
    )
    def k(dst_hbm, out_hbm, dst_v, ones_v, acc_sh, sem):
        cid = lax.axis_index("c")
        sid = lax.axis_index("s")
        wid = cid * NS + sid
        pltpu.sync_copy(dst_hbm.at[wid], dst_v)

        zero16 = jnp.zeros((16,), jnp.float32)

        @pl.loop(0, CH)
        def _(r):
            ones_v[r, :] = zero16

        @pl.loop(0, RPT // CH)
        def _(i):
            pltpu.sync_copy(ones_v, acc_sh.at[pl.ds(sid * RPT + i * CH, CH)])

        one16 = jnp.full((16,), 1.0, jnp.float32)

        @pl.loop(0, CH)
        def _(r):
            ones_v[r, :] = one16

        plsc.subcore_barrier()

        @pl.loop(0, NCH)
        def _(j):
            pltpu.sync_copy(ones_v, acc_sh.at[dst_v.at[j]], add=True)

        plsc.subcore_barrier()
        pltpu.sync_copy(acc_sh.at[pl.ds(sid * RPT, RPT)],
                        out_hbm.at[cid, pl.ds(sid * RPT, RPT)])

    return k(dst_t)


def _sc_aggregate(hs, src_t, dst_t):
    @pl.kernel(
        out_type=jax.ShapeDtypeStruct((NC, N_PAD, D), jnp.float32),
        mesh=_mesh,
        compiler_params=_sc_params,
        scratch_types=[
            pltpu.VMEM((NCH, CH), jnp.int32),
            pltpu.VMEM((NCH, CH), jnp.int32),
            pltpu.VMEM((CH, D), jnp.float32),
            pltpu.VMEM((CH, D), jnp.float32),
            pltpu.VMEM_SHARED((N_PAD, D), jnp.float32),
            pltpu.SemaphoreType.DMA,
            pltpu.SemaphoreType.DMA,
        ],
    )
    def k(hs_hbm, src_hbm, dst_hbm, out_hbm,
          src_v, dst_v, rows_a, rows_b, acc_sh, sem_a, sem_b):
        cid = lax.axis_index("c")
        sid = lax.axis_index("s")
        wid = cid * NS + sid
        pltpu.sync_copy(src_hbm.at[wid], src_v)
        pltpu.sync_copy(dst_hbm.at[wid], dst_v)

        zero16 = jnp.zeros((16,), jnp.float32)

        @pl.loop(0, CH)
        def _(r):
            @pl.loop(0, D // 16)
            def _(c):
                rows_a[r, pl.ds(c * 16, 16)] = zero16

        @pl.loop(0, RPT // CH)
        def _(i):
            pltpu.sync_copy(rows_a, acc_sh.at[pl.ds(sid * RPT + i * CH, CH)])

        plsc.subcore_barrier()

        def gather(j, buf, sem):
            pltpu.async_copy(hs_hbm.at[src_v.at[j]], buf, sem)

        def wait(j, buf, sem):
            pltpu.make_async_copy(hs_hbm.at[src_v.at[j]], buf, sem).wait()

        def scat(j, buf):
            pltpu.sync_copy(buf, acc_sh.at[dst_v.at[j]], add=True)

        gather(0, rows_a, sem_a)

        @pl.loop(0, NCH - 2, step=2)
        def _(j):
            gather(j + 1, rows_b, sem_b)
            wait(j, rows_a, sem_a)
            scat(j, rows_a)
            gather(j + 2, rows_a, sem_a)
            wait(j + 1, rows_b, sem_b)
            scat(j + 1, rows_b)

        wait(NCH - 1, rows_a, sem_a)
        scat(NCH - 1, rows_a)

        plsc.subcore_barrier()
        pltpu.sync_copy(acc_sh.at[pl.ds(sid * RPT, RPT)],
                        out_hbm.at[cid, pl.ds(sid * RPT, RPT)])

    return k(hs, src_t, dst_t)


def _tc_matmul(x, W):
    def body(x_ref, w_ref, o_ref):
        o_ref[...] = jnp.dot(x_ref[...], w_ref[...],
                             preferred_element_type=jnp.float32)
    return pl.pallas_call(
        body, out_shape=jax.ShapeDtypeStruct((x.shape[0], W.shape[1]),
                                             jnp.float32))(x, W)


def _tc_scale(h, degp):
    def body(h_ref, degp_ref, hs_ref, dinv_ref):
        deg = degp_ref[0, :N, 0:1] + degp_ref[1, :N, 0:1]
        dinv = lax.rsqrt(1.0 + deg)
        dinv_ref[...] = dinv
        hs_ref[...] = h_ref[...] * dinv
    return pl.pallas_call(
        body,
        out_shape=(jax.ShapeDtypeStruct((N, D), jnp.float32),
                   jax.ShapeDtypeStruct((N, 1), jnp.float32)))(h, degp)


def _tc_layer_out(p, hs, dinv, b, W):
    def body(p_ref, hs_ref, dinv_ref, b_ref, w_ref, o_ref):
        agg = p_ref[0, :N, :] + p_ref[1, :N, :] + hs_ref[...]
        out = jnp.maximum(dinv_ref[...] * agg + b_ref[...], 0.0)
        o_ref[...] = dinv_ref[...] * jnp.dot(out, w_ref[...],
                                             preferred_element_type=jnp.float32)
    return pl.pallas_call(
        body, out_shape=jax.ShapeDtypeStruct((N, D), jnp.float32))(
            p, hs, dinv, b, W)


def _tc_head(q, hs, dinv, b, batch2d, Wlin, blin):
    def body(q_ref, hs_ref, dinv_ref, b_ref, batch_ref,
             wl_ref, bl_ref, o_ref):
        agg = q_ref[0, :N, :] + q_ref[1, :N, :] + hs_ref[...]
        out = jnp.maximum(dinv_ref[...] * agg + b_ref[...], 0.0)
        gids = lax.broadcasted_iota(jnp.int32, (NG, N), 0)
        P = (gids == batch_ref[...]).astype(jnp.float32)
        sums = jnp.dot(P, out, preferred_element_type=jnp.float32)
        counts = jnp.sum(P, axis=1, keepdims=True)
        pooled = sums / jnp.maximum(counts, 1.0)
        o_ref[...] = jnp.dot(pooled, wl_ref[...],
                             preferred_element_type=jnp.float32) + bl_ref[...]
    return pl.pallas_call(
        body, out_shape=jax.ShapeDtypeStruct((NG, NCLS), jnp.float32))(
            q, hs, dinv, b, batch2d, Wlin, blin)


@jax.jit
def kernel(x, edge_index, batch, W1, b1, W2, b2, Wlin, blin):
    src_t = edge_index[0].reshape(NW, NCH, CH)
    dst_t = edge_index[1].reshape(NW, NCH, CH)
    batch2d = batch.reshape(1, N)
    b1r = b1.reshape(1, D)
    b2r = b2.reshape(1, D)

    degp = _sc_degree(dst_t)
    h1 = _tc_matmul(x, W1)
    hs1, dinv = _tc_scale(h1, degp)

    p = _sc_aggregate(hs1, src_t, dst_t)
    hs2 = _tc_layer_out(p, hs1, dinv, b1r, W2)

    q = _sc_aggregate(hs2, src_t, dst_t)
    return _tc_head(q, hs2, dinv, b2r, batch2d, Wlin, blin)
